# baseline (speedup 1.0000x reference)
import functools

import jax
import jax.numpy as jnp
from jax import lax
from jax.experimental import pallas as pl
from jax.experimental.pallas import tpu as pltpu
from jax.experimental.pallas import tpu_sc as plsc


_BM = 256
_BN = 4096
_SUB = 1
_BS = _BN // _SUB
_K = 4


def _topk_body(qn_ref, mn_ref, q_ref, m_ref, out_ref, sv_ref, si_ref):
    j = pl.program_id(0)
    i = pl.program_id(1)
    nj = pl.num_programs(0)

    row = pl.ds(i * _BM, _BM)

    q = q_ref[...]
    qn = qn_ref[...]
    q2 = qn ** 2
    pos = lax.broadcasted_iota(jnp.int32, (_BM, _BS), 1)
    neg_inf = jnp.float32(-jnp.inf)

    all_v, all_i = [], []
    for h in range(_SUB):
        m = m_ref[pl.ds(h * _BS, _BS), :]
        s = lax.dot_general(q, m, (((1,), (1,)), ((), ())),
                            preferred_element_type=jnp.float32)

        mn = mn_ref[:, pl.ds(h * _BS, _BS)]
        m2 = mn ** 2
        sq_dist = q2 + m2 - 2.0 * s
        dotprod = -sq_dist + q2 + m2
        dotprod = dotprod / 2.0
        dists = 1.0 - dotprod / (qn * mn)
        v = -dists

        bvals, bidx = [], []
        work = v
        for t in range(_K):
            mt = jnp.max(work, axis=1, keepdims=True)
            at = jnp.min(jnp.where(work == mt, pos, _BS), axis=1,
                         keepdims=True)
            bvals.append(mt)
            bidx.append(at + (j * _BN + h * _BS))
            if t < _K - 1:
                work = jnp.where(pos == at, neg_inf, work)

        all_v.extend(bvals)
        all_i.extend(bidx)

    blk_v = jnp.concatenate(all_v, axis=1)
    blk_i = jnp.concatenate(all_i, axis=1)
    sv_ref[j, row, :] = blk_v
    si_ref[j, row, :] = blk_i
    out_ref[...] = blk_i[:, :_K]

    @pl.when(j == nj - 1)
    def _merge():
        cv = jnp.concatenate([sv_ref[jj, row, :] for jj in range(nj)],
                             axis=1)
        ci = jnp.concatenate([si_ref[jj, row, :] for jj in range(nj)],
                             axis=1)
        nc = _SUB * nj * _K
        posc = lax.broadcasted_iota(jnp.int32, (_BM, nc), 1)
        ni_ = []
        for t in range(_K):
            mt = jnp.max(cv, axis=1, keepdims=True)
            at = jnp.min(jnp.where(cv == mt, posc, nc), axis=1,
                         keepdims=True)
            ni_.append(jnp.max(jnp.where(posc == at, ci, -1), axis=1,
                               keepdims=True))
            if t < _K - 1:
                cv = jnp.where(posc == at, neg_inf, cv)
        out_ref[...] = jnp.concatenate(ni_, axis=1)


def _tc_topk(query_seq, matching_set, qn, mn):
    M, D = query_seq.shape
    N = matching_set.shape[0]
    ni, nj = M // _BM, N // _BN
    return pl.pallas_call(
        _topk_body,
        grid=(nj, ni),
        in_specs=[
            pl.BlockSpec((_BM, 1), lambda j, i: (i, 0)),
            pl.BlockSpec((1, _BN), lambda j, i: (0, j)),
            pl.BlockSpec((_BM, D), lambda j, i: (i, 0)),
            pl.BlockSpec((_BN, D), lambda j, i: (j, 0)),
        ],
        out_specs=pl.BlockSpec((_BM, _K), lambda j, i: (i, 0)),
        out_shape=jax.ShapeDtypeStruct((M, _K), jnp.int32),
        scratch_shapes=[
            pltpu.VMEM((nj, M, _SUB * _K), jnp.float32),
            pltpu.VMEM((nj, M, _SUB * _K), jnp.int32),
        ],
        compiler_params=pltpu.CompilerParams(
            dimension_semantics=("arbitrary", "arbitrary")),
    )(qn.reshape(M, 1), mn.reshape(1, N), query_seq, matching_set)


_NC, _NS = 2, 16
_NW = _NC * _NS
_CH_Q = 4


def _sc_gather_sum(idx_flat, synth_set, M, D, topk):
    ipw = idx_flat.shape[0] // _NW
    qpw = M // _NW
    nch = qpw // _CH_Q
    ch_i = _CH_Q * _K
    scale = jnp.float32(1.0 / topk)

    mesh = plsc.VectorSubcoreMesh(core_axis_name="c", subcore_axis_name="s")

    @functools.partial(
        pl.kernel,
        mesh=mesh,
        out_type=jax.ShapeDtypeStruct((M, D), jnp.float32),
        scratch_types=[
            pltpu.VMEM((ipw,), jnp.int32),
            pltpu.VMEM((ch_i, D), jnp.float32),
            pltpu.VMEM((ch_i, D), jnp.float32),
            pltpu.VMEM((_CH_Q, D), jnp.float32),
            pltpu.SemaphoreType.DMA,
            pltpu.SemaphoreType.DMA,
        ],
    )
    def k(idx_hbm, synth_hbm, out_hbm, idx_v, rows0, rows1, acc_v,
          sem0, sem1):
        wid = lax.axis_index("s") * _NC + lax.axis_index("c")
        base_i = wid * ipw
        base_q = wid * qpw
        pltpu.sync_copy(idx_hbm.at[pl.ds(base_i, ipw)], idx_v)
        bufs = (rows0, rows1)
        sems = (sem0, sem1)

        def start(ch):
            return pltpu.async_copy(
                synth_hbm.at[idx_v.at[pl.ds(ch * ch_i, ch_i)]],
                bufs[ch % 2], sems[ch % 2])

        handles = [start(0), None]
        for ch in range(nch):
            handles[ch % 2].wait()
            if ch + 1 < nch:
                handles[(ch + 1) % 2] = start(ch + 1)
            rows_v = bufs[ch % 2]
            for q in range(_CH_Q):
                def dbody(dd, _, q=q, rows_v=rows_v):
                    sl = pl.ds(dd * 16, 16)
                    r = (rows_v[_K * q, sl] + rows_v[_K * q + 1, sl]
                         + rows_v[_K * q + 2, sl] + rows_v[_K * q + 3, sl])
                    acc_v[q, sl] = r * scale
                    return 0
                lax.fori_loop(0, D // 16, dbody, 0)
            pltpu.sync_copy(acc_v,
                            out_hbm.at[pl.ds(base_q + ch * _CH_Q, _CH_Q)])

    return k(idx_flat, synth_set)


def kernel(query_seq, matching_set, synth_set, topk):
    M, D = query_seq.shape
    N = matching_set.shape[0]
    qn = jnp.linalg.norm(query_seq, ord=2, axis=-1)
    mn = jnp.linalg.norm(matching_set, ord=2, axis=-1)
    indices = _tc_topk(query_seq, matching_set, qn, mn)
    return _sc_gather_sum(indices.reshape(M * _K), synth_set, M, D, topk)


# speedup vs baseline: 1.0206x; 1.0206x over previous
import functools

import jax
import jax.numpy as jnp
from jax import lax
from jax.experimental import pallas as pl
from jax.experimental.pallas import tpu as pltpu
from jax.experimental.pallas import tpu_sc as plsc


_BM = 256
_BN = 4096
_SUB = 1
_BS = _BN // _SUB
_K = 4


def _topk_body(qn_ref, mn_ref, q_ref, m_ref, out_ref, sv_ref, si_ref):
    j = pl.program_id(0)
    i = pl.program_id(1)
    nj = pl.num_programs(0)

    row = pl.ds(i * _BM, _BM)

    q = q_ref[...]
    qn = qn_ref[...]
    q2 = qn ** 2
    pos = lax.broadcasted_iota(jnp.int32, (_BM, _BS), 1)
    pos_inf = jnp.float32(jnp.inf)

    all_v, all_i = [], []
    for h in range(_SUB):
        m = m_ref[pl.ds(h * _BS, _BS), :]
        s = lax.dot_general(q, m, (((1,), (1,)), ((), ())),
                            preferred_element_type=jnp.float32)

        mn = mn_ref[:, pl.ds(h * _BS, _BS)]
        m2 = mn ** 2
        sq_dist = q2 + m2 - 2.0 * s
        dotprod = -sq_dist + q2 + m2
        dotprod = dotprod / 2.0
        dists = 1.0 - dotprod / (qn * mn)

        bvals, bidx = [], []
        work = dists
        for t in range(_K):
            mt = jnp.min(work, axis=1, keepdims=True)
            at = jnp.min(jnp.where(work == mt, pos, _BS), axis=1,
                         keepdims=True)
            bvals.append(mt)
            bidx.append(at + (j * _BN + h * _BS))
            if t < _K - 1:
                work = jnp.where(pos == at, pos_inf, work)

        all_v.extend(bvals)
        all_i.extend(bidx)

    blk_v = jnp.concatenate(all_v, axis=1)
    blk_i = jnp.concatenate(all_i, axis=1)
    sv_ref[j, row, :] = blk_v
    si_ref[j, row, :] = blk_i
    out_ref[...] = blk_i[:, :_K]

    @pl.when(j == nj - 1)
    def _merge():
        cv = jnp.concatenate([sv_ref[jj, row, :] for jj in range(nj)],
                             axis=1)
        ci = jnp.concatenate([si_ref[jj, row, :] for jj in range(nj)],
                             axis=1)
        nc = _SUB * nj * _K
        posc = lax.broadcasted_iota(jnp.int32, (_BM, nc), 1)
        ni_ = []
        for t in range(_K):
            mt = jnp.min(cv, axis=1, keepdims=True)
            at = jnp.min(jnp.where(cv == mt, posc, nc), axis=1,
                         keepdims=True)
            ni_.append(jnp.max(jnp.where(posc == at, ci, -1), axis=1,
                               keepdims=True))
            if t < _K - 1:
                cv = jnp.where(posc == at, pos_inf, cv)
        out_ref[...] = jnp.concatenate(ni_, axis=1)


def _tc_topk(query_seq, matching_set, qn, mn):
    M, D = query_seq.shape
    N = matching_set.shape[0]
    ni, nj = M // _BM, N // _BN
    return pl.pallas_call(
        _topk_body,
        grid=(nj, ni),
        in_specs=[
            pl.BlockSpec((_BM, 1), lambda j, i: (i, 0)),
            pl.BlockSpec((1, _BN), lambda j, i: (0, j)),
            pl.BlockSpec((_BM, D), lambda j, i: (i, 0)),
            pl.BlockSpec((_BN, D), lambda j, i: (j, 0)),
        ],
        out_specs=pl.BlockSpec((_BM, _K), lambda j, i: (i, 0)),
        out_shape=jax.ShapeDtypeStruct((M, _K), jnp.int32),
        scratch_shapes=[
            pltpu.VMEM((nj, M, _SUB * _K), jnp.float32),
            pltpu.VMEM((nj, M, _SUB * _K), jnp.int32),
        ],
        compiler_params=pltpu.CompilerParams(
            dimension_semantics=("arbitrary", "arbitrary")),
    )(qn.reshape(M, 1), mn.reshape(1, N), query_seq, matching_set)


_NC, _NS = 2, 16
_NW = _NC * _NS
_CH_Q = 8


def _sc_gather_sum(idx_flat, synth_set, M, D, topk):
    ipw = idx_flat.shape[0] // _NW
    qpw = M // _NW
    nch = qpw // _CH_Q
    ch_i = _CH_Q * _K
    scale = jnp.float32(1.0 / topk)

    mesh = plsc.VectorSubcoreMesh(core_axis_name="c", subcore_axis_name="s")

    @functools.partial(
        pl.kernel,
        mesh=mesh,
        out_type=jax.ShapeDtypeStruct((M, D), jnp.float32),
        scratch_types=[
            pltpu.VMEM((ipw,), jnp.int32),
            pltpu.VMEM((ch_i, D), jnp.float32),
            pltpu.VMEM((_CH_Q, D), jnp.float32),
            pltpu.SemaphoreType.DMA,
        ],
    )
    def k(idx_hbm, synth_hbm, out_hbm, idx_v, rows_v, acc_v, sem):
        wid = lax.axis_index("s") * _NC + lax.axis_index("c")
        base_i = wid * ipw
        base_q = wid * qpw
        pltpu.sync_copy(idx_hbm.at[pl.ds(base_i, ipw)], idx_v)
        for ch in range(nch):
            pltpu.async_copy(
                synth_hbm.at[idx_v.at[pl.ds(ch * ch_i, ch_i)]],
                rows_v, sem).wait()
            for q in range(_CH_Q):
                def dbody(dd, _, q=q):
                    sl = pl.ds(dd * 16, 16)
                    r = (rows_v[_K * q, sl] + rows_v[_K * q + 1, sl]
                         + rows_v[_K * q + 2, sl] + rows_v[_K * q + 3, sl])
                    acc_v[q, sl] = r * scale
                    return 0
                lax.fori_loop(0, D // 16, dbody, 0)
            pltpu.sync_copy(acc_v,
                            out_hbm.at[pl.ds(base_q + ch * _CH_Q, _CH_Q)])

    return k(idx_flat, synth_set)


def kernel(query_seq, matching_set, synth_set, topk):
    M, D = query_seq.shape
    N = matching_set.shape[0]
    qn = jnp.linalg.norm(query_seq, ord=2, axis=-1)
    mn = jnp.linalg.norm(matching_set, ord=2, axis=-1)
    indices = _tc_topk(query_seq, matching_set, qn, mn)
    return _sc_gather_sum(indices.reshape(M * _K), synth_set, M, D, topk)
